# baseline (speedup 1.0000x reference)
import functools
import math

import jax
import jax.numpy as jnp
from jax import lax
from jax.experimental import pallas as pl
from jax.experimental.pallas import tpu as pltpu
from jax.experimental.pallas import tpu_sc as plsc

BLKQ, BLKK = 64, 32
TOPK_FRAC = 0.1
LUTPAD = 16


def _stats_kernel(q_ref, k_ref, v_ref, kvsum_ref, ksum_ref, ps_ref,
                  *, mb, nb):
    q2 = q_ref[0]
    k2 = k_ref[0]
    v2 = v_ref[0]
    l, dd = q2.shape
    d = dd // 2
    vb2 = v2.astype(jnp.bfloat16)
    km2 = jnp.mean(k2, axis=0, keepdims=True)
    pq2 = jnp.mean(q2.reshape(mb, BLKQ, dd), axis=1)
    pk2 = jnp.mean(k2.reshape(nb, BLKK, dd), axis=1) - km2
    for hh in range(2):
        ps_ref[0, hh] = jax.lax.dot_general(
            pq2[:, hh * d:(hh + 1) * d], pk2[:, hh * d:(hh + 1) * d],
            (((1,), (1,)), ((), ())), precision=jax.lax.Precision.HIGHEST,
            preferred_element_type=jnp.float32)
    mx = jnp.max(k2, axis=1, keepdims=True)
    e = jnp.exp(k2 - mx)
    r0 = jax.lax.broadcasted_iota(jnp.int32, (dd, dd), 0) // d
    r1 = jax.lax.broadcasted_iota(jnp.int32, (dd, dd), 1) // d
    blkones = (r0 == r1).astype(jnp.bfloat16)
    rs = jax.lax.dot_general(e.astype(jnp.bfloat16), blkones,
                             (((1,), (0,)), ((), ())),
                             preferred_element_type=jnp.float32)
    kf = e / rs
    ksum2 = jnp.sum(kf, axis=0, keepdims=True)
    ksum_ref[0, 0] = ksum2[:, :d]
    ksum_ref[0, 1] = ksum2[:, d:]
    kvp = jax.lax.dot_general(kf.astype(jnp.bfloat16), vb2,
                              (((0,), (0,)), ((), ())),
                              preferred_element_type=jnp.float32)
    kvsum_ref[0, 0] = kvp[:d, :d]
    kvsum_ref[0, 1] = kvp[d:, d:]


def _sc_topk(ps_hbm, lut_hbm, ps_v, lut_v, *, rows_per, nb, topk):
    info = plsc.get_sparse_core_info()
    nc = info.num_cores
    wid = lax.axis_index("s") * nc + lax.axis_index("c")
    base = wid * rows_per
    pltpu.sync_copy(ps_hbm.at[pl.ds(base, rows_per)], ps_v)
    nvec = nb // 16
    lanes = jnp.arange(16, dtype=jnp.int32)
    perms = [jnp.bitwise_xor(lanes, sh) for sh in (1, 2, 4, 8)]

    gdn = lax.GatherDimensionNumbers(offset_dims=(), collapsed_slice_dims=(0,),
                                     start_index_map=(0,))

    def _shuf(x, p):
        return lax.gather(x, p[:, None], gdn, (1,),
                          mode=lax.GatherScatterMode.PROMISE_IN_BOUNDS)

    def row_body(r, _):
        vecs = [ps_v[r, pl.ds(j * 16, 16)] for j in range(nvec)]
        out = jnp.zeros((16,), jnp.int32)
        for t in range(topk):
            m = vecs[0]
            for j in range(1, nvec):
                m = jnp.maximum(m, vecs[j])
            for p in perms:
                m = jnp.maximum(m, _shuf(m, p))
            idx = jnp.full((16,), nb, jnp.int32)
            for j in range(nvec):
                idx = jnp.minimum(idx, jnp.where(vecs[j] >= m,
                                                 lanes + 16 * j, nb))
            for p in perms:
                idx = jnp.minimum(idx, _shuf(idx, p))
            out = jnp.where(lanes == t, idx, out)
            for j in range(nvec):
                vecs[j] = jnp.where(lanes + 16 * j == idx,
                                    jnp.float32(-3.0e38), vecs[j])
        lut_v[r] = out
        return _

    lax.fori_loop(0, rows_per, row_body, 0)
    pltpu.sync_copy(lut_v, lut_hbm.at[pl.ds(base, rows_per)])


def _topk_lut(ps, nb, topk):
    rows = ps.shape[0] * ps.shape[1]
    info = plsc.get_sparse_core_info()
    nw = info.num_cores * info.num_subcores
    rows_per = rows // nw
    mesh = plsc.VectorSubcoreMesh(core_axis_name="c", subcore_axis_name="s")
    f = functools.partial(
        pl.kernel,
        mesh=mesh,
        out_type=jax.ShapeDtypeStruct((rows, LUTPAD), jnp.int32),
        scratch_types=[
            pltpu.VMEM((rows_per, nb), jnp.float32),
            pltpu.VMEM((rows_per, LUTPAD), jnp.int32),
        ],
    )(functools.partial(_sc_topk, rows_per=rows_per, nb=nb, topk=topk))
    return f(ps.reshape(rows, nb)).reshape(ps.shape[0], ps.shape[1], LUTPAD)


def _attn_kernel(lut_ref, q_ref, k_ref, v_ref, kvsum_ref, ksum_ref,
                 w_ref, b_ref, o_ref, kc_scr, vc_scr, s_scr, p_scr, os_scr,
                 *, topk, scale, mg, nh):
    ib = pl.program_id(0)
    jh = pl.program_id(1)
    jm = pl.program_id(2)
    d = w_ref.shape[0]
    for hh in range(2):
        hidx = ib * nh + jh * 2 + hh
        lo = hh * d
        for g in range(mg):
            m = jm * mg + g
            for t in range(topk):
                idx = lut_ref[hidx, m, t]
                off = idx * BLKK
                kc_scr[hh, g, t * BLKK:(t + 1) * BLKK, :] = (
                    k_ref[0, pl.ds(off, BLKK), lo:lo + d].astype(jnp.bfloat16))
                vc_scr[hh, g, t * BLKK:(t + 1) * BLKK, :] = (
                    v_ref[0, pl.ds(off, BLKK), lo:lo + d].astype(jnp.bfloat16))
    for hh in range(2):
        qs = (q_ref[0, :, hh * d:(hh + 1) * d] * scale).astype(jnp.bfloat16)
        for g in range(mg):
            s_scr[hh, g] = jax.lax.dot_general(
                qs[g * BLKQ:(g + 1) * BLKQ, :], kc_scr[hh, g],
                (((1,), (1,)), ((), ())), preferred_element_type=jnp.float32)
    sa = s_scr[...].reshape(2 * mg * BLKQ, topk * BLKK)
    e = jnp.exp(sa - jnp.max(sa, axis=1, keepdims=True))
    pn = e / jnp.sum(e, axis=1, keepdims=True)
    p_scr[...] = pn.astype(jnp.bfloat16).reshape(2, mg, BLKQ, topk * BLKK)
    for hh in range(2):
        for g in range(mg):
            os_scr[hh, g * BLKQ:(g + 1) * BLKQ, :] = jax.lax.dot_general(
                p_scr[hh, g], vc_scr[hh, g], (((1,), (0,)), ((), ())),
                preferred_element_type=jnp.float32)
    halves = []
    for hh in range(2):
        qall = q_ref[0, :, hh * d:(hh + 1) * d]
        qf = jax.nn.softmax(qall, axis=-1)
        denom = jnp.sum(qf * ksum_ref[0, hh], axis=1, keepdims=True) + 1e-6
        num = jax.lax.dot_general(qf, kvsum_ref[0, hh], (((1,), (0,)), ((), ())),
                                  preferred_element_type=jnp.float32)
        o_l = num / denom
        o_l = jax.lax.dot_general(o_l, w_ref[...], (((1,), (1,)), ((), ())),
                                  preferred_element_type=jnp.float32) + b_ref[0]
        halves.append(o_l + os_scr[hh])
    o_ref[0] = jnp.concatenate(halves, axis=1)


def kernel(q, k, v, W_proj, b_proj):
    b, l, h, d = q.shape
    bh = b * h
    mb = l // BLKQ
    nb = l // BLKK
    topk = min(nb, int(TOPK_FRAC * nb))
    scale = 1.0 / math.sqrt(d)

    q3 = q.reshape(b, l, h * d)
    k3 = k.reshape(b, l, h * d)
    v3 = v.reshape(b, l, h * d)

    kvsum, ksum, ps = pl.pallas_call(
        functools.partial(_stats_kernel, mb=mb, nb=nb),
        grid=(b, h // 2),
        in_specs=[
            pl.BlockSpec((1, l, 2 * d), lambda i, j: (i, 0, j)),
            pl.BlockSpec((1, l, 2 * d), lambda i, j: (i, 0, j)),
            pl.BlockSpec((1, l, 2 * d), lambda i, j: (i, 0, j)),
        ],
        out_specs=[
            pl.BlockSpec((1, 2, d, d), lambda i, j: (i, j, 0, 0)),
            pl.BlockSpec((1, 2, 1, d), lambda i, j: (i, j, 0, 0)),
            pl.BlockSpec((1, 2, mb, nb), lambda i, j: (i, j, 0, 0)),
        ],
        out_shape=[
            jax.ShapeDtypeStruct((b, h, d, d), jnp.float32),
            jax.ShapeDtypeStruct((b, h, 1, d), jnp.float32),
            jax.ShapeDtypeStruct((b, h, mb, nb), jnp.float32),
        ],
        compiler_params=pltpu.CompilerParams(
            dimension_semantics=("arbitrary", "arbitrary")),
    )(q3, k3, v3)

    lut = _topk_lut(ps.reshape(bh, mb, nb), nb, topk)

    mg = 16
    out = pl.pallas_call(
        functools.partial(_attn_kernel, topk=topk, scale=scale, mg=mg, nh=h),
        grid=(b, h // 2, mb // mg),
        in_specs=[
            pl.BlockSpec(memory_space=pltpu.SMEM),
            pl.BlockSpec((1, mg * BLKQ, 2 * d), lambda i, j, m: (i, m, j)),
            pl.BlockSpec((1, l, 2 * d), lambda i, j, m: (i, 0, j)),
            pl.BlockSpec((1, l, 2 * d), lambda i, j, m: (i, 0, j)),
            pl.BlockSpec((1, 2, d, d), lambda i, j, m: (i, j, 0, 0)),
            pl.BlockSpec((1, 2, 1, d), lambda i, j, m: (i, j, 0, 0)),
            pl.BlockSpec((d, d), lambda i, j, m: (0, 0)),
            pl.BlockSpec((1, d), lambda i, j, m: (0, 0)),
        ],
        out_specs=pl.BlockSpec((1, mg * BLKQ, 2 * d), lambda i, j, m: (i, m, j)),
        out_shape=jax.ShapeDtypeStruct((b, l, h * d), jnp.float32),
        scratch_shapes=[
            pltpu.VMEM((2, mg, topk * BLKK, d), jnp.bfloat16),
            pltpu.VMEM((2, mg, topk * BLKK, d), jnp.bfloat16),
            pltpu.VMEM((2, mg, BLKQ, topk * BLKK), jnp.float32),
            pltpu.VMEM((2, mg, BLKQ, topk * BLKK), jnp.bfloat16),
            pltpu.VMEM((2, mg * BLKQ, d), jnp.float32),
        ],
        compiler_params=pltpu.CompilerParams(
            dimension_semantics=("arbitrary", "arbitrary", "arbitrary")),
    )(lut, q3, k3, v3, kvsum, ksum, W_proj, b_proj.reshape(1, d))

    return out.reshape(b, l, h, d)


# speedup vs baseline: 1.1034x; 1.1034x over previous
import functools
import math

import jax
import jax.numpy as jnp
from jax import lax
from jax.experimental import pallas as pl
from jax.experimental.pallas import tpu as pltpu
from jax.experimental.pallas import tpu_sc as plsc

BLKQ, BLKK = 64, 32
TOPK_FRAC = 0.1
LUTPAD = 16


def _stats_kernel(q_ref, k_ref, v_ref, qt_ref, kb_ref, vb_ref,
                  kvsum_ref, ksum_ref, ps_ref, *, mb, nb):
    q2 = q_ref[0]
    k2 = k_ref[0]
    v2 = v_ref[0]
    l, dd = q2.shape
    d = dd // 2
    qt_ref[0, 0] = q2[:, :d]
    qt_ref[0, 1] = q2[:, d:]
    kb2 = k2.astype(jnp.bfloat16)
    vb2 = v2.astype(jnp.bfloat16)
    kb_ref[0, 0] = kb2[:, :d]
    kb_ref[0, 1] = kb2[:, d:]
    vb_ref[0, 0] = vb2[:, :d]
    vb_ref[0, 1] = vb2[:, d:]
    km2 = jnp.mean(k2, axis=0, keepdims=True)
    pq2 = jnp.mean(q2.reshape(mb, BLKQ, dd), axis=1)
    pk2 = jnp.mean(k2.reshape(nb, BLKK, dd), axis=1) - km2
    for hh in range(2):
        ps_ref[0, hh] = jax.lax.dot_general(
            pq2[:, hh * d:(hh + 1) * d], pk2[:, hh * d:(hh + 1) * d],
            (((1,), (1,)), ((), ())), precision=jax.lax.Precision.HIGHEST,
            preferred_element_type=jnp.float32)
    mx = jnp.max(k2, axis=1, keepdims=True)
    e = jnp.exp(k2 - mx)
    r0 = jax.lax.broadcasted_iota(jnp.int32, (dd, dd), 0) // d
    r1 = jax.lax.broadcasted_iota(jnp.int32, (dd, dd), 1) // d
    blkones = (r0 == r1).astype(jnp.bfloat16)
    rs = jax.lax.dot_general(e.astype(jnp.bfloat16), blkones,
                             (((1,), (0,)), ((), ())),
                             preferred_element_type=jnp.float32)
    kf = e / rs
    ksum2 = jnp.sum(kf, axis=0, keepdims=True)
    ksum_ref[0, 0] = ksum2[:, :d]
    ksum_ref[0, 1] = ksum2[:, d:]
    kvp = jax.lax.dot_general(kf.astype(jnp.bfloat16), vb2,
                              (((0,), (0,)), ((), ())),
                              preferred_element_type=jnp.float32)
    kvsum_ref[0, 0] = kvp[:d, :d]
    kvsum_ref[0, 1] = kvp[d:, d:]


def _sc_topk(ps_hbm, lut_hbm, ps_v, lut_v, *, rows_per, nb, topk):
    info = plsc.get_sparse_core_info()
    nc = info.num_cores
    wid = lax.axis_index("s") * nc + lax.axis_index("c")
    base = wid * rows_per
    pltpu.sync_copy(ps_hbm.at[pl.ds(base, rows_per)], ps_v)
    nvec = nb // 16
    lanes = jnp.arange(16, dtype=jnp.int32)
    perms = [jnp.bitwise_xor(lanes, sh) for sh in (1, 2, 4, 8)]
    gdn = lax.GatherDimensionNumbers(offset_dims=(), collapsed_slice_dims=(0,),
                                     start_index_map=(0,))

    def _shuf(x, p):
        return lax.gather(x, p[:, None], gdn, (1,),
                          mode=lax.GatherScatterMode.PROMISE_IN_BOUNDS)

    def row_body(r, _):
        vecs = [ps_v[r, pl.ds(j * 16, 16)] for j in range(nvec)]
        out = jnp.zeros((16,), jnp.int32)
        for t in range(topk):
            m = vecs[0]
            for j in range(1, nvec):
                m = jnp.maximum(m, vecs[j])
            for p in perms:
                m = jnp.maximum(m, _shuf(m, p))
            idx = jnp.full((16,), nb, jnp.int32)
            for j in range(nvec):
                idx = jnp.minimum(idx, jnp.where(vecs[j] >= m,
                                                 lanes + 16 * j, nb))
            for p in perms:
                idx = jnp.minimum(idx, _shuf(idx, p))
            out = jnp.where(lanes == t, idx, out)
            for j in range(nvec):
                vecs[j] = jnp.where(lanes + 16 * j == idx,
                                    jnp.float32(-3.0e38), vecs[j])
        lut_v[r] = out
        return _

    lax.fori_loop(0, rows_per, row_body, 0)
    pltpu.sync_copy(lut_v, lut_hbm.at[pl.ds(base, rows_per)])


def _topk_lut(ps, nb, topk):
    rows = ps.shape[0] * ps.shape[1]
    info = plsc.get_sparse_core_info()
    nw = info.num_cores * info.num_subcores
    rows_per = rows // nw
    mesh = plsc.VectorSubcoreMesh(core_axis_name="c", subcore_axis_name="s")
    f = functools.partial(
        pl.kernel,
        mesh=mesh,
        out_type=jax.ShapeDtypeStruct((rows, LUTPAD), jnp.int32),
        scratch_types=[
            pltpu.VMEM((rows_per, nb), jnp.float32),
            pltpu.VMEM((rows_per, LUTPAD), jnp.int32),
        ],
    )(functools.partial(_sc_topk, rows_per=rows_per, nb=nb, topk=topk))
    return f(ps.reshape(rows, nb)).reshape(ps.shape[0], ps.shape[1], LUTPAD)


def _attn_kernel(lut_ref, q_ref, k_ref, v_ref, kvsum_ref, ksum_ref,
                 w_ref, b_ref, o_ref, kc_scr, vc_scr, s_scr, p_scr, os_scr,
                 *, topk, scale, mg):
    h = pl.program_id(0)
    jg = pl.program_id(1)
    for g in range(mg):
        m = jg * mg + g
        for t in range(topk):
            idx = lut_ref[h, m, t]
            off = idx * BLKK
            kc_scr[g, t * BLKK:(t + 1) * BLKK, :] = k_ref[0, pl.ds(off, BLKK), :]
            vc_scr[g, t * BLKK:(t + 1) * BLKK, :] = v_ref[0, pl.ds(off, BLKK), :]
    qall = q_ref[0]
    qs = (qall * scale).astype(jnp.bfloat16)
    for g in range(mg):
        s_scr[g] = jax.lax.dot_general(
            qs[g * BLKQ:(g + 1) * BLKQ, :], kc_scr[g],
            (((1,), (1,)), ((), ())), preferred_element_type=jnp.float32)
    sa = s_scr[...].reshape(mg * BLKQ, topk * BLKK)
    e = jnp.exp(sa - jnp.max(sa, axis=1, keepdims=True))
    pn = e / jnp.sum(e, axis=1, keepdims=True)
    p_scr[...] = pn.astype(jnp.bfloat16).reshape(mg, BLKQ, topk * BLKK)
    for g in range(mg):
        os_scr[g * BLKQ:(g + 1) * BLKQ, :] = jax.lax.dot_general(
            p_scr[g], vc_scr[g], (((1,), (0,)), ((), ())),
            preferred_element_type=jnp.float32)
    qf = jax.nn.softmax(qall, axis=-1)
    denom = jnp.sum(qf * ksum_ref[0], axis=1, keepdims=True) + 1e-6
    num = jax.lax.dot_general(qf, kvsum_ref[0], (((1,), (0,)), ((), ())),
                              preferred_element_type=jnp.float32)
    o_l = num / denom
    o_l = jax.lax.dot_general(o_l, w_ref[...], (((1,), (1,)), ((), ())),
                              preferred_element_type=jnp.float32) + b_ref[0]
    o_ref[0] = o_l + os_scr[...]


def kernel(q, k, v, W_proj, b_proj):
    b, l, h, d = q.shape
    bh = b * h
    mb = l // BLKQ
    nb = l // BLKK
    topk = min(nb, int(TOPK_FRAC * nb))
    scale = 1.0 / math.sqrt(d)

    q3 = q.reshape(b, l, h * d)
    k3 = k.reshape(b, l, h * d)
    v3 = v.reshape(b, l, h * d)

    qt, kb, vb, kvsum, ksum, ps = pl.pallas_call(
        functools.partial(_stats_kernel, mb=mb, nb=nb),
        grid=(b, h // 2),
        in_specs=[
            pl.BlockSpec((1, l, 2 * d), lambda i, j: (i, 0, j)),
            pl.BlockSpec((1, l, 2 * d), lambda i, j: (i, 0, j)),
            pl.BlockSpec((1, l, 2 * d), lambda i, j: (i, 0, j)),
        ],
        out_specs=[
            pl.BlockSpec((1, 2, l, d), lambda i, j: (i, j, 0, 0)),
            pl.BlockSpec((1, 2, l, d), lambda i, j: (i, j, 0, 0)),
            pl.BlockSpec((1, 2, l, d), lambda i, j: (i, j, 0, 0)),
            pl.BlockSpec((1, 2, d, d), lambda i, j: (i, j, 0, 0)),
            pl.BlockSpec((1, 2, 1, d), lambda i, j: (i, j, 0, 0)),
            pl.BlockSpec((1, 2, mb, nb), lambda i, j: (i, j, 0, 0)),
        ],
        out_shape=[
            jax.ShapeDtypeStruct((b, h, l, d), jnp.float32),
            jax.ShapeDtypeStruct((b, h, l, d), jnp.bfloat16),
            jax.ShapeDtypeStruct((b, h, l, d), jnp.bfloat16),
            jax.ShapeDtypeStruct((b, h, d, d), jnp.float32),
            jax.ShapeDtypeStruct((b, h, 1, d), jnp.float32),
            jax.ShapeDtypeStruct((b, h, mb, nb), jnp.float32),
        ],
        compiler_params=pltpu.CompilerParams(
            dimension_semantics=("arbitrary", "arbitrary")),
    )(q3, k3, v3)
    qt = qt.reshape(bh, l, d)
    kb = kb.reshape(bh, l, d)
    vb = vb.reshape(bh, l, d)
    kvsum = kvsum.reshape(bh, d, d)
    ksum = ksum.reshape(bh, 1, d)

    lut = _topk_lut(ps.reshape(bh, mb, nb), nb, topk)

    mg = 16
    out = pl.pallas_call(
        functools.partial(_attn_kernel, topk=topk, scale=scale, mg=mg),
        grid=(bh, mb // mg),
        in_specs=[
            pl.BlockSpec(memory_space=pltpu.SMEM),
            pl.BlockSpec((1, mg * BLKQ, d), lambda i, j: (i, j, 0)),
            pl.BlockSpec((1, l, d), lambda i, j: (i, 0, 0)),
            pl.BlockSpec((1, l, d), lambda i, j: (i, 0, 0)),
            pl.BlockSpec((1, d, d), lambda i, j: (i, 0, 0)),
            pl.BlockSpec((1, 1, d), lambda i, j: (i, 0, 0)),
            pl.BlockSpec((d, d), lambda i, j: (0, 0)),
            pl.BlockSpec((1, d), lambda i, j: (0, 0)),
        ],
        out_specs=pl.BlockSpec((1, mg * BLKQ, d), lambda i, j: (i, j, 0)),
        out_shape=jax.ShapeDtypeStruct((bh, l, d), jnp.float32),
        scratch_shapes=[
            pltpu.VMEM((mg, topk * BLKK, d), jnp.bfloat16),
            pltpu.VMEM((mg, topk * BLKK, d), jnp.bfloat16),
            pltpu.VMEM((mg, BLKQ, topk * BLKK), jnp.float32),
            pltpu.VMEM((mg, BLKQ, topk * BLKK), jnp.bfloat16),
            pltpu.VMEM((mg * BLKQ, d), jnp.float32),
        ],
        compiler_params=pltpu.CompilerParams(
            dimension_semantics=("arbitrary", "arbitrary")),
    )(lut, qt, kb, vb, kvsum, ksum, W_proj, b_proj.reshape(1, d))

    return jnp.transpose(out.reshape(b, h, l, d), (0, 2, 1, 3))
